# baseline (speedup 1.0000x reference)
import functools

import jax
import jax.numpy as jnp
from jax import lax
from jax.experimental import pallas as pl
from jax.experimental.pallas import tpu as pltpu
from jax.experimental.pallas import tpu_sc as plsc

_BETA = 0.3
_NUM_EXAMP = 1000000
_NUM_CLASSES = 100
_CPAD = 104
_BATCH = 16384
_NC = 2
_NS = 16
_NW = _NC * _NS
_WG = 256
_NG = (_NUM_EXAMP + _WG - 1) // _WG
_LAST_G = _NG - 1
_LAST_W = _NUM_EXAMP - _LAST_G * _WG
_GPW = (_NG + _NW - 1) // _NW
_TAIL_B = _LAST_G * _WG
_L = 16
_SEG = 4096
_OUT_ROWS = _BATCH + _NW * _L


def _iota16():
    return lax.iota(jnp.int32, _L)


def _sc_sweep_body(idx_hbm, ttab_hbm, ttail_hbm, out_hbm, lb, lp, mb, mp,
                   slab_a, slab_b, rowbuf, slabsem, rowsem):
    c = lax.axis_index("c")
    s = lax.axis_index("s")
    wid = s * _NC + c
    g_lo = wid * _GPW
    g_hi = jnp.minimum(g_lo + _GPW, _LAST_G)
    b_lo = g_lo * _WG
    b_hi = jnp.minimum(g_hi * _WG, _TAIL_B)

    def _build_list(lo, hi):
        def _p1_chunk(ch, cnt):
            pltpu.sync_copy(idx_hbm.at[pl.ds(ch * _SEG, _SEG)], mb.at[pl.ds(0, _SEG)])

            def _p1_scan(k, cnt):
                v = mb[pl.ds(k * _L, _L)]
                m = (v >= lo) & (v < hi)
                pos = ch * _SEG + k * _L + _iota16()
                key = jnp.where(m, _iota16(), jnp.int32(_L))
                lb[pl.ds(cnt, _L)] = plsc.sort_key_val(key, v)[1]
                lp[pl.ds(cnt, _L)] = plsc.sort_key_val(key, pos)[1]
                return cnt + plsc.all_reduce_population_count(m)[0]

            return lax.fori_loop(0, _SEG // _L, _p1_scan, cnt)

        return lax.fori_loop(0, _BATCH // _SEG, _p1_chunk, 0)

    cnt = _build_list(b_lo, b_hi)
    lb[pl.ds(cnt, _L)] = jnp.zeros((_L,), jnp.int32) - 1

    n_my = g_hi - g_lo
    slabs = (slab_a, slab_b)

    def _slab_start(g, sl):
        pltpu.make_async_copy(
            ttab_hbm.at[pl.ds(0, _NUM_CLASSES), pl.ds(g * _WG, _WG)],
            sl[0],
            slabsem.at[sl[1]],
        ).start()

    def _slab_wait(sl):
        pltpu.make_async_copy(
            ttab_hbm.at[pl.ds(0, _NUM_CLASSES), pl.ds(0, _WG)],
            sl[0],
            slabsem.at[sl[1]],
        ).wait()

    @pl.when(g_lo < g_hi)
    def _():
        _slab_start(g_lo, (slab_a, 0))

    def _do_group(i, ec, sl, sl_next):
        g = g_lo + i
        valid_g = i < n_my

        @pl.when(valid_g)
        def _():
            _slab_wait(sl)

        @pl.when(valid_g & (i + 1 < n_my))
        def _():
            _slab_start(g + 1, sl_next)

        def _wave(w, ec):
            seg = jnp.minimum(_SEG, cnt - w * _SEG)

            def _scan(q, mcnt):
                base = w * _SEG + q * _L
                vb = lb[pl.ds(base, _L)]
                m = (vb >> 8) == g
                npc = plsc.all_reduce_population_count(m)[0]

                @pl.when(npc > 0)
                def _():
                    vp = lp[pl.ds(base, _L)]
                    key = jnp.where(m, _iota16(), jnp.int32(_L))
                    mb[pl.ds(mcnt, _L)] = plsc.sort_key_val(key, vb)[1]
                    mp[pl.ds(mcnt, _L)] = plsc.sort_key_val(key, vp)[1]

                return mcnt + npc

            nq = (seg + _L - 1) >> 4
            mcnt = lax.fori_loop(0, nq, _scan, 0)

            def _extract(q2, ec):
                for t in range(_L):
                    @pl.when(ec >= 1)
                    def _():
                        pltpu.make_async_copy(
                            rowbuf.at[t], out_hbm.at[0], rowsem.at[t]
                        ).wait()

                v16b = mb[pl.ds(q2 * _L, _L)]
                v16p = mp[pl.ds(q2 * _L, _L)]
                for t in range(_L):
                    val = (q2 * _L + t) < mcnt
                    b = v16b[t]
                    pos = jnp.where(val, v16p[t], _BATCH + wid * _L + t)
                    lane = b & (_WG - 1)
                    lvec = jnp.zeros((_L,), jnp.int32) + lane
                    for k in range(7):
                        cvec = jnp.minimum(k * _L + _iota16(), _NUM_CLASSES - 1)
                        vals = plsc.load_gather(sl[0], [cvec, lvec])
                        rowbuf[t, pl.ds(k * _L, _L)] = vals
                    pltpu.make_async_copy(
                        rowbuf.at[t], out_hbm.at[pos], rowsem.at[t]
                    ).start()
                return ec + 1

            nq2 = (mcnt + _L - 1) >> 4
            return lax.fori_loop(0, nq2, _extract, ec)

        nwaves = jnp.where(valid_g, (cnt + _SEG - 1) >> 12, 0)
        return lax.fori_loop(0, nwaves, _wave, ec)

    def _pair_body(ip, ec):
        ec = _do_group(ip * 2, ec, (slab_a, 0), (slab_b, 1))
        ec = _do_group(ip * 2 + 1, ec, (slab_b, 1), (slab_a, 0))
        return ec

    ec = lax.fori_loop(0, (_GPW + 1) // 2, _pair_body, 0)

    t_lo = jnp.where(wid == _NW - 1, _TAIL_B, 0)
    t_hi = jnp.where(wid == _NW - 1, _NUM_EXAMP, 0)
    cnt2 = _build_list(t_lo, t_hi)

    def _tail_extract(q2, ec):
        for t in range(_L):
            @pl.when(ec >= 1)
            def _():
                pltpu.make_async_copy(
                    rowbuf.at[t], out_hbm.at[0], rowsem.at[t]
                ).wait()

        v16b = lb[pl.ds(q2 * _L, _L)]
        v16p = lp[pl.ds(q2 * _L, _L)]
        for t in range(_L):
            val = (q2 * _L + t) < cnt2
            tb = jnp.where(val, v16b[t] - _TAIL_B, 0)
            pos = jnp.where(val, v16p[t], _BATCH + wid * _L + t)
            pltpu.sync_copy(ttail_hbm.at[tb],
                            rowbuf.at[t, pl.ds(0, _NUM_CLASSES)])
            pltpu.make_async_copy(
                rowbuf.at[t], out_hbm.at[pos], rowsem.at[t]
            ).start()
        return ec + 1

    ec = lax.fori_loop(0, (cnt2 + _L - 1) >> 4, _tail_extract, ec)

    for t in range(_L):
        @pl.when(ec >= 1)
        def _():
            pltpu.make_async_copy(
                rowbuf.at[t], out_hbm.at[0], rowsem.at[t]
            ).wait()


@functools.cache
def _sc_sweep():
    return pl.kernel(
        _sc_sweep_body,
        out_type=jax.ShapeDtypeStruct((_OUT_ROWS, 128), jnp.float32),
        mesh=plsc.VectorSubcoreMesh(core_axis_name="c", subcore_axis_name="s"),
        scratch_types=[
            pltpu.VMEM((_BATCH + _L,), jnp.int32),
            pltpu.VMEM((_BATCH + _L,), jnp.int32),
            pltpu.VMEM((_SEG + _L,), jnp.int32),
            pltpu.VMEM((_SEG + _L,), jnp.int32),
            pltpu.VMEM((_NUM_CLASSES, _WG), jnp.float32),
            pltpu.VMEM((_NUM_CLASSES, _WG), jnp.float32),
            pltpu.VMEM((_L, 128), jnp.float32),
            pltpu.SemaphoreType.DMA((2,)),
            pltpu.SemaphoreType.DMA((_L,)),
        ],
        compiler_params=pltpu.CompilerParams(needs_layout_passes=False),
    )


def _loss_body(out_ref, lab_ref, old_ref, loss_ref):
    x = out_ref[...]
    lab = lab_ref[...]
    old = old_ref[...][:, :_NUM_CLASSES]
    m = jnp.max(x, axis=1, keepdims=True)
    e = jnp.exp(x - m)
    s = jnp.sum(e, axis=1, keepdims=True)
    p = e / s
    pc = jnp.clip(p, 0.0001, 1.0 - 0.0001)
    q = pc / jnp.sum(pc, axis=1, keepdims=True)
    new = _BETA * old + (1.0 - _BETA) * q
    dot = jnp.sum(new * pc, axis=1)
    cols = lax.broadcasted_iota(jnp.int32, x.shape, 1)
    xl = jnp.sum(jnp.where(cols == lab[:, None], x, 0.0), axis=1)
    ce = jnp.log(s[:, 0]) + m[:, 0] - xl
    loss_ref[...] = ce + 5.0 * jnp.log(1.0 - dot)


def _loss_call(output, label, old_pad, rows_per_block=2048):
    grid = _BATCH // rows_per_block
    return pl.pallas_call(
        _loss_body,
        grid=(grid,),
        in_specs=[
            pl.BlockSpec((rows_per_block, _NUM_CLASSES), lambda i: (i, 0)),
            pl.BlockSpec((rows_per_block,), lambda i: (i,)),
            pl.BlockSpec((rows_per_block, 128), lambda i: (i, 0)),
        ],
        out_specs=pl.BlockSpec((rows_per_block,), lambda i: (i,)),
        out_shape=jax.ShapeDtypeStruct((_BATCH,), jnp.float32),
    )(output, label, old_pad)


def kernel(index, output, label, target):
    ttail = lax.slice(target, (_TAIL_B, 0), (_NUM_EXAMP, _NUM_CLASSES))
    old_pad = _sc_sweep()(index, target.T, ttail)
    return _loss_call(output, label, old_pad)


# speedup vs baseline: 1.1561x; 1.1561x over previous
import functools

import jax
import jax.numpy as jnp
from jax import lax
from jax.experimental import pallas as pl
from jax.experimental.pallas import tpu as pltpu
from jax.experimental.pallas import tpu_sc as plsc

_BETA = 0.3
_NUM_EXAMP = 1000000
_NUM_CLASSES = 100
_CPAD = 104
_BATCH = 16384
_NC = 2
_NS = 16
_NW = _NC * _NS
_WG = 256
_NG = (_NUM_EXAMP + _WG - 1) // _WG
_LAST_G = _NG - 1
_LAST_W = _NUM_EXAMP - _LAST_G * _WG
_GPW = (_NG + _NW - 1) // _NW
_TAIL_B = _LAST_G * _WG
_L = 16
_SEG = 4096
_OUT_ROWS = _BATCH + _NW * _L


def _iota16():
    return lax.iota(jnp.int32, _L)


def _sc_sweep_body(idx_hbm, ttab_hbm, ttail_hbm, out_hbm, lb, lp, mb, mp,
                   slab_a, slab_b, rowbuf, slabsem, rowsem):
    c = lax.axis_index("c")
    s = lax.axis_index("s")
    wid = s * _NC + c
    g_lo = wid * _GPW
    g_hi = jnp.minimum(g_lo + _GPW, _LAST_G)
    b_lo = g_lo * _WG
    b_hi = jnp.minimum(g_hi * _WG, _TAIL_B)

    def _build_list(lo, hi):
        def _p1_chunk(ch, cnt):
            pltpu.sync_copy(idx_hbm.at[pl.ds(ch * _SEG, _SEG)], mb.at[pl.ds(0, _SEG)])

            def _p1_scan(k, cnt):
                v = mb[pl.ds(k * _L, _L)]
                m = (v >= lo) & (v < hi)
                pos = ch * _SEG + k * _L + _iota16()
                key = jnp.where(m, _iota16(), jnp.int32(_L))
                lb[pl.ds(cnt, _L)] = plsc.sort_key_val(key, v)[1]
                lp[pl.ds(cnt, _L)] = plsc.sort_key_val(key, pos)[1]
                return cnt + plsc.all_reduce_population_count(m)[0]

            return lax.fori_loop(0, _SEG // _L, _p1_scan, cnt)

        return lax.fori_loop(0, _BATCH // _SEG, _p1_chunk, 0)

    n_my = g_hi - g_lo

    def _slab_start(g, sl):
        pltpu.make_async_copy(
            ttab_hbm.at[pl.ds(0, _NUM_CLASSES), pl.ds(g * _WG, _WG)],
            sl[0],
            slabsem.at[sl[1]],
        ).start()

    def _slab_wait(sl):
        pltpu.make_async_copy(
            ttab_hbm.at[pl.ds(0, _NUM_CLASSES), pl.ds(0, _WG)],
            sl[0],
            slabsem.at[sl[1]],
        ).wait()

    @pl.when(g_lo < g_hi)
    def _():
        _slab_start(g_lo, (slab_a, 0))

    cnt = _build_list(b_lo, b_hi)

    def _do_group(i, ec, sl, sl_next):
        g = g_lo + i
        valid_g = i < n_my

        @pl.when(valid_g)
        def _():
            _slab_wait(sl)

        @pl.when(valid_g & (i + 1 < n_my))
        def _():
            _slab_start(g + 1, sl_next)

        def _wave(w, ec):
            seg = jnp.minimum(_SEG, cnt - w * _SEG)

            def _scan(q, mcnt):
                base = w * _SEG + q * _L
                vb = lb[pl.ds(base, _L)]
                vp = lp[pl.ds(base, _L)]
                valid = (q * _L + _iota16()) < seg
                m = ((vb >> 8) == g) & valid
                key = jnp.where(m, _iota16(), jnp.int32(_L))
                mb[pl.ds(mcnt, _L)] = plsc.sort_key_val(key, vb)[1]
                mp[pl.ds(mcnt, _L)] = plsc.sort_key_val(key, vp)[1]
                return mcnt + plsc.all_reduce_population_count(m)[0]

            nq = (seg + _L - 1) >> 4
            mcnt = lax.fori_loop(0, nq, _scan, 0)

            def _extract(q2, ec):
                for t in range(_L):
                    @pl.when(ec >= 1)
                    def _():
                        pltpu.make_async_copy(
                            rowbuf.at[t], out_hbm.at[0], rowsem.at[t]
                        ).wait()

                v16b = mb[pl.ds(q2 * _L, _L)]
                v16p = mp[pl.ds(q2 * _L, _L)]
                for t in range(_L):
                    val = (q2 * _L + t) < mcnt
                    b = v16b[t]
                    pos = jnp.where(val, v16p[t], _BATCH + wid * _L + t)
                    lane = b & (_WG - 1)
                    lvec = jnp.zeros((_L,), jnp.int32) + lane
                    for k in range(7):
                        cvec = jnp.minimum(k * _L + _iota16(), _NUM_CLASSES - 1)
                        vals = plsc.load_gather(sl[0], [cvec, lvec])
                        rowbuf[t, pl.ds(k * _L, _L)] = vals
                    pltpu.make_async_copy(
                        rowbuf.at[t], out_hbm.at[pos], rowsem.at[t]
                    ).start()
                return ec + 1

            nq2 = (mcnt + _L - 1) >> 4
            return lax.fori_loop(0, nq2, _extract, ec)

        nwaves = jnp.where(valid_g, (cnt + _SEG - 1) >> 12, 0)
        return lax.fori_loop(0, nwaves, _wave, ec)

    def _pair_body(ip, ec):
        ec = _do_group(ip * 2, ec, (slab_a, 0), (slab_b, 1))
        ec = _do_group(ip * 2 + 1, ec, (slab_b, 1), (slab_a, 0))
        return ec

    ec = lax.fori_loop(0, (_GPW + 1) // 2, _pair_body, 0)

    t_lo = jnp.where(wid == _NW - 1, _TAIL_B, 0)
    t_hi = jnp.where(wid == _NW - 1, _NUM_EXAMP, 0)
    cnt2 = _build_list(t_lo, t_hi)

    def _tail_extract(q2, ec):
        for t in range(_L):
            @pl.when(ec >= 1)
            def _():
                pltpu.make_async_copy(
                    rowbuf.at[t], out_hbm.at[0], rowsem.at[t]
                ).wait()

        v16b = lb[pl.ds(q2 * _L, _L)]
        v16p = lp[pl.ds(q2 * _L, _L)]
        for t in range(_L):
            val = (q2 * _L + t) < cnt2
            tb = jnp.where(val, v16b[t] - _TAIL_B, 0)
            pos = jnp.where(val, v16p[t], _BATCH + wid * _L + t)
            pltpu.sync_copy(ttail_hbm.at[tb],
                            rowbuf.at[t, pl.ds(0, _NUM_CLASSES)])
            pltpu.make_async_copy(
                rowbuf.at[t], out_hbm.at[pos], rowsem.at[t]
            ).start()
        return ec + 1

    ec = lax.fori_loop(0, (cnt2 + _L - 1) >> 4, _tail_extract, ec)

    for t in range(_L):
        @pl.when(ec >= 1)
        def _():
            pltpu.make_async_copy(
                rowbuf.at[t], out_hbm.at[0], rowsem.at[t]
            ).wait()


@functools.cache
def _sc_sweep():
    return pl.kernel(
        _sc_sweep_body,
        out_type=jax.ShapeDtypeStruct((_OUT_ROWS, 128), jnp.float32),
        mesh=plsc.VectorSubcoreMesh(core_axis_name="c", subcore_axis_name="s"),
        scratch_types=[
            pltpu.VMEM((_BATCH + _L,), jnp.int32),
            pltpu.VMEM((_BATCH + _L,), jnp.int32),
            pltpu.VMEM((_SEG + _L,), jnp.int32),
            pltpu.VMEM((_SEG + _L,), jnp.int32),
            pltpu.VMEM((_NUM_CLASSES, _WG), jnp.float32),
            pltpu.VMEM((_NUM_CLASSES, _WG), jnp.float32),
            pltpu.VMEM((_L, 128), jnp.float32),
            pltpu.SemaphoreType.DMA((2,)),
            pltpu.SemaphoreType.DMA((_L,)),
        ],
        compiler_params=pltpu.CompilerParams(needs_layout_passes=False),
    )


def _loss_body(out_ref, lab_ref, old_ref, loss_ref):
    x = out_ref[...]
    lab = lab_ref[...]
    old = old_ref[...][:, :_NUM_CLASSES]
    m = jnp.max(x, axis=1, keepdims=True)
    e = jnp.exp(x - m)
    s = jnp.sum(e, axis=1, keepdims=True)
    p = e / s
    pc = jnp.clip(p, 0.0001, 1.0 - 0.0001)
    q = pc / jnp.sum(pc, axis=1, keepdims=True)
    new = _BETA * old + (1.0 - _BETA) * q
    dot = jnp.sum(new * pc, axis=1)
    cols = lax.broadcasted_iota(jnp.int32, x.shape, 1)
    xl = jnp.sum(jnp.where(cols == lab[:, None], x, 0.0), axis=1)
    ce = jnp.log(s[:, 0]) + m[:, 0] - xl
    loss_ref[...] = ce + 5.0 * jnp.log(1.0 - dot)


def _loss_call(output, label, old_pad, rows_per_block=2048):
    grid = _BATCH // rows_per_block
    return pl.pallas_call(
        _loss_body,
        grid=(grid,),
        in_specs=[
            pl.BlockSpec((rows_per_block, _NUM_CLASSES), lambda i: (i, 0)),
            pl.BlockSpec((rows_per_block,), lambda i: (i,)),
            pl.BlockSpec((rows_per_block, 128), lambda i: (i, 0)),
        ],
        out_specs=pl.BlockSpec((rows_per_block,), lambda i: (i,)),
        out_shape=jax.ShapeDtypeStruct((_BATCH,), jnp.float32),
    )(output, label, old_pad)


def kernel(index, output, label, target):
    ttail = lax.slice(target, (_TAIL_B, 0), (_NUM_EXAMP, _NUM_CLASSES))
    old_pad = _sc_sweep()(index, target.T, ttail)
    return _loss_call(output, label, old_pad)


# speedup vs baseline: 1.1566x; 1.0004x over previous
import functools

import jax
import jax.numpy as jnp
from jax import lax
from jax.experimental import pallas as pl
from jax.experimental.pallas import tpu as pltpu
from jax.experimental.pallas import tpu_sc as plsc

_BETA = 0.3
_NUM_EXAMP = 1000000
_NUM_CLASSES = 100
_CPAD = 104
_BATCH = 16384
_NC = 2
_NS = 16
_NW = _NC * _NS
_WG = 256
_NG = (_NUM_EXAMP + _WG - 1) // _WG
_LAST_G = _NG - 1
_LAST_W = _NUM_EXAMP - _LAST_G * _WG
_GPW = (_NG + _NW - 1) // _NW
_TAIL_B = _LAST_G * _WG
_L = 16
_SEG = 4096
_OUT_ROWS = _BATCH + _NW * _L


def _iota16():
    return lax.iota(jnp.int32, _L)


def _sc_sweep_body(idx_hbm, ttab_hbm, ttail_hbm, out_hbm, lb, lp, mb, mp,
                   slab_a, slab_b, rowbuf, slabsem, rowsem):
    c = lax.axis_index("c")
    s = lax.axis_index("s")
    wid = s * _NC + c
    g_lo = wid * _GPW
    g_hi = jnp.minimum(g_lo + _GPW, _LAST_G)
    b_lo = g_lo * _WG
    b_hi = jnp.minimum(g_hi * _WG, _TAIL_B)

    def _build_list(lo, hi):
        def _p1_chunk(ch, cnt):
            pltpu.sync_copy(idx_hbm.at[pl.ds(ch * _SEG, _SEG)], mb.at[pl.ds(0, _SEG)])

            def _p1_scan(k, cnt):
                v = mb[pl.ds(k * _L, _L)]
                m = (v >= lo) & (v < hi)
                pos = ch * _SEG + k * _L + _iota16()
                key = jnp.where(m, _iota16(), jnp.int32(_L))
                lb[pl.ds(cnt, _L)] = plsc.sort_key_val(key, v)[1]
                lp[pl.ds(cnt, _L)] = plsc.sort_key_val(key, pos)[1]
                return cnt + plsc.all_reduce_population_count(m)[0]

            return lax.fori_loop(0, _SEG // _L, _p1_scan, cnt)

        return lax.fori_loop(0, _BATCH // _SEG, _p1_chunk, 0)

    n_my = g_hi - g_lo

    def _slab_start(g, sl):
        pltpu.make_async_copy(
            ttab_hbm.at[pl.ds(0, _NUM_CLASSES), pl.ds(g * _WG, _WG)],
            sl[0],
            slabsem.at[sl[1]],
        ).start()

    def _slab_wait(sl):
        pltpu.make_async_copy(
            ttab_hbm.at[pl.ds(0, _NUM_CLASSES), pl.ds(0, _WG)],
            sl[0],
            slabsem.at[sl[1]],
        ).wait()

    @pl.when(g_lo < g_hi)
    def _():
        _slab_start(g_lo, (slab_a, 0))

    cnt = _build_list(b_lo, b_hi)

    def _do_group(i, ec, sl, sl_next):
        g = g_lo + i
        valid_g = i < n_my

        @pl.when(valid_g)
        def _():
            _slab_wait(sl)

        @pl.when(valid_g & (i + 1 < n_my))
        def _():
            _slab_start(g + 1, sl_next)

        def _wave(w, ec):
            seg = jnp.minimum(_SEG, cnt - w * _SEG)

            def _scan(q, mcnt):
                base = w * _SEG + q * _L
                vb = lb[pl.ds(base, _L)]
                vp = lp[pl.ds(base, _L)]
                valid = (q * _L + _iota16()) < seg
                m = ((vb >> 8) == g) & valid
                key = jnp.where(m, _iota16(), jnp.int32(_L))
                mb[pl.ds(mcnt, _L)] = plsc.sort_key_val(key, vb)[1]
                mp[pl.ds(mcnt, _L)] = plsc.sort_key_val(key, vp)[1]
                return mcnt + plsc.all_reduce_population_count(m)[0]

            nq = (seg + _L - 1) >> 4
            mcnt = lax.fori_loop(0, nq, _scan, 0)

            def _extract(q2, ec):
                for t in range(_L):
                    @pl.when(ec >= 1)
                    def _():
                        pltpu.make_async_copy(
                            rowbuf.at[t], out_hbm.at[0], rowsem.at[t]
                        ).wait()

                v16b = mb[pl.ds(q2 * _L, _L)]
                v16p = mp[pl.ds(q2 * _L, _L)]
                for t in range(_L):
                    val = (q2 * _L + t) < mcnt
                    b = v16b[t]
                    pos = jnp.where(val, v16p[t], _BATCH + wid * _L + t)
                    lane = b & (_WG - 1)
                    lvec = jnp.zeros((_L,), jnp.int32) + lane
                    for k in range(7):
                        cvec = jnp.minimum(k * _L + _iota16(), _NUM_CLASSES - 1)
                        vals = plsc.load_gather(sl[0], [cvec, lvec])
                        rowbuf[t, pl.ds(k * _L, _L)] = vals
                    pltpu.make_async_copy(
                        rowbuf.at[t], out_hbm.at[pos], rowsem.at[t]
                    ).start()
                return ec + 1

            nq2 = (mcnt + _L - 1) >> 4
            return lax.fori_loop(0, nq2, _extract, ec)

        nwaves = jnp.where(valid_g, (cnt + _SEG - 1) >> 12, 0)
        return lax.fori_loop(0, nwaves, _wave, ec)

    def _pair_body(ip, ec):
        ec = _do_group(ip * 2, ec, (slab_a, 0), (slab_b, 1))
        ec = _do_group(ip * 2 + 1, ec, (slab_b, 1), (slab_a, 0))
        return ec

    ec = lax.fori_loop(0, (_GPW + 1) // 2, _pair_body, 0)

    t_lo = jnp.where(wid == _NW - 1, _TAIL_B, 0)
    t_hi = jnp.where(wid == _NW - 1, _NUM_EXAMP, 0)
    cnt2 = _build_list(t_lo, t_hi)

    def _tail_extract(q2, ec):
        for t in range(_L):
            @pl.when(ec >= 1)
            def _():
                pltpu.make_async_copy(
                    rowbuf.at[t], out_hbm.at[0], rowsem.at[t]
                ).wait()

        v16b = lb[pl.ds(q2 * _L, _L)]
        v16p = lp[pl.ds(q2 * _L, _L)]
        for t in range(_L):
            val = (q2 * _L + t) < cnt2
            tb = jnp.where(val, v16b[t] - _TAIL_B, 0)
            pos = jnp.where(val, v16p[t], _BATCH + wid * _L + t)
            pltpu.sync_copy(ttail_hbm.at[tb],
                            rowbuf.at[t, pl.ds(0, _NUM_CLASSES)])
            pltpu.make_async_copy(
                rowbuf.at[t], out_hbm.at[pos], rowsem.at[t]
            ).start()
        return ec + 1

    ec = lax.fori_loop(0, (cnt2 + _L - 1) >> 4, _tail_extract, ec)

    for t in range(_L):
        @pl.when(ec >= 1)
        def _():
            pltpu.make_async_copy(
                rowbuf.at[t], out_hbm.at[0], rowsem.at[t]
            ).wait()


@functools.cache
def _sc_sweep():
    return pl.kernel(
        _sc_sweep_body,
        out_type=jax.ShapeDtypeStruct((_OUT_ROWS, 128), jnp.float32),
        mesh=plsc.VectorSubcoreMesh(core_axis_name="c", subcore_axis_name="s"),
        scratch_types=[
            pltpu.VMEM((_BATCH + _L,), jnp.int32),
            pltpu.VMEM((_BATCH + _L,), jnp.int32),
            pltpu.VMEM((_SEG + _L,), jnp.int32),
            pltpu.VMEM((_SEG + _L,), jnp.int32),
            pltpu.VMEM((_NUM_CLASSES, _WG), jnp.float32),
            pltpu.VMEM((_NUM_CLASSES, _WG), jnp.float32),
            pltpu.VMEM((_L, 128), jnp.float32),
            pltpu.SemaphoreType.DMA((2,)),
            pltpu.SemaphoreType.DMA((_L,)),
        ],
        compiler_params=pltpu.CompilerParams(needs_layout_passes=False),
    )


def _loss_body(out_ref, lab_ref, old_ref, loss_ref):
    x = out_ref[...]
    lab = lab_ref[...]
    old = old_ref[...][:, :_NUM_CLASSES]
    m = jnp.max(x, axis=1, keepdims=True)
    e = jnp.exp(x - m)
    s = jnp.sum(e, axis=1, keepdims=True)
    p = e / s
    pc = jnp.clip(p, 0.0001, 1.0 - 0.0001)
    q = pc / jnp.sum(pc, axis=1, keepdims=True)
    new = _BETA * old + (1.0 - _BETA) * q
    dot = jnp.sum(new * pc, axis=1)
    cols = lax.broadcasted_iota(jnp.int32, x.shape, 1)
    xl = jnp.sum(jnp.where(cols == lab[:, None], x, 0.0), axis=1)
    ce = jnp.log(s[:, 0]) + m[:, 0] - xl
    loss_ref[...] = ce + 5.0 * jnp.log(1.0 - dot)


def _loss_call(output, label, old_pad, rows_per_block=4096):
    grid = _BATCH // rows_per_block
    return pl.pallas_call(
        _loss_body,
        grid=(grid,),
        in_specs=[
            pl.BlockSpec((rows_per_block, _NUM_CLASSES), lambda i: (i, 0)),
            pl.BlockSpec((rows_per_block,), lambda i: (i,)),
            pl.BlockSpec((rows_per_block, 128), lambda i: (i, 0)),
        ],
        out_specs=pl.BlockSpec((rows_per_block,), lambda i: (i,)),
        out_shape=jax.ShapeDtypeStruct((_BATCH,), jnp.float32),
    )(output, label, old_pad)


def kernel(index, output, label, target):
    ttail = lax.slice(target, (_TAIL_B, 0), (_NUM_EXAMP, _NUM_CLASSES))
    old_pad = _sc_sweep()(index, target.T, ttail)
    return _loss_call(output, label, old_pad)
